# Initial kernel scaffold; baseline (speedup 1.0000x reference)
import jax
import jax.numpy as jnp
from jax.experimental import pallas as pl


def kernel(x, W1, b1, W2, b2, W3, b3, Wl, bl, Wm1, bm1, Wm2, bm2, Wm3, bm3):
    raise NotImplementedError("write your pallas kernel here")


# speedup vs baseline: 5.0138x; 5.0138x over previous
import functools

import jax
import jax.numpy as jnp
from jax import lax
from jax.experimental import pallas as pl
from jax.experimental.pallas import tpu as pltpu
from jax.experimental.pallas import tpu_sc as plsc

N = 10000
K = 20
NPAD = 10240
RB = 128
NBLK = NPAD // RB

NC = 2
NS = 16
NW = NC * NS
NODES_PER_W = NPAD // NW
SC_NB = 8


def _knn_ag_body(x_blk, xT, Wa, Wb, b, idx_out, a_out, g_out, dist, xx_scr):
    i = pl.program_id(0)

    @pl.when(i == 0)
    def _():
        xx_scr[...] = jnp.sum(xT[...] * xT[...], axis=0, keepdims=True)

    xb = x_blk[...]
    g = jnp.dot(xb, Wb[...], preferred_element_type=jnp.float32)
    a = jnp.dot(xb, Wa[...], preferred_element_type=jnp.float32) + b[...] - g
    g_out[...] = g
    a_out[...] = a

    xx_r = jnp.sum(xb * xb, axis=1, keepdims=True)
    d = xx_r + xx_scr[...] - 2.0 * jnp.dot(
        xb, xT[...], preferred_element_type=jnp.float32)
    col = lax.broadcasted_iota(jnp.int32, (RB, NPAD), 1)
    dist[...] = jnp.where(col < N, d, jnp.inf)

    for t in range(K):
        dd = dist[...]
        m = jnp.min(dd, axis=1, keepdims=True)
        sel = jnp.min(jnp.where(dd == m, col, NPAD),
                      axis=1, keepdims=True)
        idx_out[:, t:t + 1] = sel
        if t < K - 1:
            dist[...] = jnp.where(col == sel, jnp.inf, dd)


def _knn_ag_call(dp, Dp, interpret=False):
    return pl.pallas_call(
        _knn_ag_body,
        grid=(NBLK,),
        in_specs=[
            pl.BlockSpec((RB, dp), lambda i: (i, 0)),
            pl.BlockSpec((dp, NPAD), lambda i: (0, 0)),
            pl.BlockSpec((dp, Dp), lambda i: (0, 0)),
            pl.BlockSpec((dp, Dp), lambda i: (0, 0)),
            pl.BlockSpec((1, Dp), lambda i: (0, 0)),
        ],
        out_specs=[
            pl.BlockSpec((RB, 128), lambda i: (i, 0)),
            pl.BlockSpec((RB, Dp), lambda i: (i, 0)),
            pl.BlockSpec((RB, Dp), lambda i: (i, 0)),
        ],
        out_shape=[
            jax.ShapeDtypeStruct((NPAD, 128), jnp.int32),
            jax.ShapeDtypeStruct((NPAD, Dp), jnp.float32),
            jax.ShapeDtypeStruct((NPAD, Dp), jnp.float32),
        ],
        scratch_shapes=[
            pltpu.VMEM((RB, NPAD), jnp.float32),
            pltpu.VMEM((1, NPAD), jnp.float32),
        ],
        interpret=interpret,
    )


def _gather_max_call(Dp):
    CH = Dp // 16
    NSB = NODES_PER_W // SC_NB
    mesh = plsc.VectorSubcoreMesh(
        core_axis_name="c", subcore_axis_name="s",
        num_cores=NC, num_subcores=NS)

    @functools.partial(
        pl.kernel,
        out_type=jax.ShapeDtypeStruct((NPAD, Dp), jnp.float32),
        mesh=mesh,
        scratch_types=[
            pltpu.VMEM((SC_NB * K,), jnp.int32),
            pltpu.VMEM((SC_NB * K, Dp), jnp.float32),
            pltpu.VMEM((SC_NB, Dp), jnp.float32),
            pltpu.VMEM((SC_NB, Dp), jnp.float32),
            pltpu.SemaphoreType.DMA,
        ],
    )
    def k(idx_hbm, g_hbm, a_hbm, out_hbm, idx_v, rows_v, a_v, o_v, sem):
        wid = lax.axis_index("s") * NC + lax.axis_index("c")
        base_node = wid * NODES_PER_W

        def body(sb, _):
            nb0 = base_node + sb * SC_NB
            pltpu.sync_copy(idx_hbm.at[pl.ds(nb0 * K, SC_NB * K)], idx_v)
            pltpu.async_copy(g_hbm.at[idx_v], rows_v, sem).wait()
            pltpu.sync_copy(a_hbm.at[pl.ds(nb0, SC_NB)], a_v)
            for n in range(SC_NB):
                for c in range(CH):
                    cs = pl.ds(c * 16, 16)
                    acc = rows_v[n * K, cs]
                    for kk in range(1, K):
                        acc = jnp.maximum(acc, rows_v[n * K + kk, cs])
                    o_v[n, cs] = jnp.maximum(a_v[n, cs] + acc, 0.0)
            pltpu.sync_copy(o_v, out_hbm.at[pl.ds(nb0, SC_NB)])
            return ()

        lax.fori_loop(0, NSB, body, ())

    return k


def _head_body(x1, x2, x3, Wl1, Wl2, Wl3, bl, Wm1, bm1, Wm2, bm2, Wm3, bm3,
               out):
    f32 = jnp.float32
    h = (jnp.dot(x1[...], Wl1[...], preferred_element_type=f32)
         + jnp.dot(x2[...], Wl2[...], preferred_element_type=f32)
         + jnp.dot(x3[...], Wl3[...], preferred_element_type=f32)
         + bl[...])
    h = jnp.maximum(h, 0.0)
    h = jnp.maximum(jnp.dot(h, Wm1[...], preferred_element_type=f32)
                    + bm1[...], 0.0)
    h = jnp.maximum(jnp.dot(h, Wm2[...], preferred_element_type=f32)
                    + bm2[...], 0.0)
    v = jnp.dot(h, Wm3[...], preferred_element_type=f32) + bm3[...]
    lane = lax.broadcasted_iota(jnp.int32, v.shape, 1)
    vm = jnp.where(lane < 40, v, -jnp.inf)
    mx = jnp.max(vm, axis=1, keepdims=True)
    lse = jnp.log(jnp.sum(jnp.exp(vm - mx), axis=1, keepdims=True)) + mx
    out[...] = v - lse


def _head_call(interpret=False):
    full = lambda r, c: pl.BlockSpec((r, c), lambda i: (0, 0))
    return pl.pallas_call(
        _head_body,
        grid=(NBLK,),
        in_specs=[
            pl.BlockSpec((RB, 128), lambda i: (i, 0)),
            pl.BlockSpec((RB, 128), lambda i: (i, 0)),
            pl.BlockSpec((RB, 256), lambda i: (i, 0)),
            full(128, 1024), full(128, 1024), full(256, 1024), full(1, 1024),
            full(1024, 256), full(1, 256),
            full(256, 128), full(1, 128),
            full(128, 128), full(1, 128),
        ],
        out_specs=pl.BlockSpec((RB, 128), lambda i: (i, 0)),
        out_shape=jax.ShapeDtypeStruct((NPAD, 128), jnp.float32),
        interpret=interpret,
    )


def _pad2(w, r, c):
    return jnp.zeros((r, c), jnp.float32).at[:w.shape[0], :w.shape[1]].set(w)


def _edge_layer(xp, W, b, d, dp, Dp):
    Wa = _pad2(W[:d], dp, Dp)
    Wb = _pad2(W[d:], dp, Dp)
    bp = _pad2(b[None, :], 1, Dp)
    idx, a, g = _knn_ag_call(dp, Dp)(xp, xp.T, Wa, Wb, bp)
    idx_flat = idx[:, :K].reshape(-1)
    return _gather_max_call(Dp)(idx_flat, g, a)


def kernel(x, W1, b1, W2, b2, W3, b3, Wl, bl, Wm1, bm1, Wm2, bm2, Wm3, bm3):
    xp = jnp.zeros((NPAD, 8), jnp.float32).at[:N, :3].set(x)
    x1 = _edge_layer(xp, W1, b1, 3, 8, 128)
    x2 = _edge_layer(x1, W2, b2, 60, 128, 128)
    x3 = _edge_layer(x2, W3, b3, 120, 128, 256)

    out = _head_call()(
        x1, x2, x3,
        _pad2(Wl[:60], 128, 1024), _pad2(Wl[60:180], 128, 1024),
        _pad2(Wl[180:], 256, 1024), bl[None, :],
        Wm1, bm1[None, :], Wm2, bm2[None, :],
        _pad2(Wm3, 128, 128), _pad2(bm3[None, :], 1, 128),
    )
    return out[:N, :40]
